# baseline (speedup 1.0000x reference)
import functools

import jax
import jax.numpy as jnp
from jax import lax
from jax.experimental import pallas as pl
from jax.experimental.pallas import tpu as pltpu
from jax.experimental.pallas import tpu_sc as plsc

_N = 10000
_E = 160000
_HS = 256
_BN = 400
_EPT = _E // 16
_K = 128
_BLK = 2000
_NB = _E // _BLK
_OWN = 320
_ARB = 17
_N2 = 10240


def _tc_pre_body(h_ref, c_ref, x_ref, uft_ref, bf_ref, wbut_ref, bbu_ref,
                 wtdt_ref, btd_ref, g_ref, pbu_ref, ptd_ref):
    f = jax.nn.sigmoid(
        jnp.dot(h_ref[...], uft_ref[...], preferred_element_type=jnp.float32)
        + bf_ref[...])
    g_ref[...] = f * c_ref[...]
    x = x_ref[...]
    pbu_ref[...] = jnp.dot(x, wbut_ref[...],
                           preferred_element_type=jnp.float32) + bbu_ref[...]
    ptd_ref[...] = jnp.dot(x, wtdt_ref[...],
                           preferred_element_type=jnp.float32) + btd_ref[...]


def _tc_post_body(pbu_ref, hsum_ref, cred_ref, ptd_ref, hp_ref, cp_ref,
                  maxe_ref, ubut_ref, utdt_ref, out_ref):
    iou_bu = pbu_ref[...] + jnp.dot(hsum_ref[...], ubut_ref[...],
                                    preferred_element_type=jnp.float32)
    i_bu = iou_bu[:, 0:_HS]
    o_bu = iou_bu[:, _HS:2 * _HS]
    u_bu = iou_bu[:, 2 * _HS:3 * _HS]
    c_bu = jax.nn.sigmoid(i_bu) * jnp.tanh(u_bu) + cred_ref[...]
    h_bu = jax.nn.sigmoid(o_bu) * jnp.tanh(c_bu)
    has = maxe_ref[...] >= 0
    hp = jnp.where(has, hp_ref[...], 0.0)
    cp = jnp.where(has, cp_ref[...], 0.0)
    iou_td = ptd_ref[...] + jnp.dot(hp, utdt_ref[...],
                                    preferred_element_type=jnp.float32)
    i_td = iou_td[:, 0:_HS]
    o_td = iou_td[:, _HS:2 * _HS]
    u_td = iou_td[:, 2 * _HS:3 * _HS]
    c_td = jax.nn.sigmoid(i_td) * jnp.tanh(u_td) + cp
    h_td = jax.nn.sigmoid(o_td) * jnp.tanh(c_td)
    out_ref[...] = jnp.concatenate([h_bu, c_bu, h_td, c_td], axis=1)


def _row_spec(width):
    return pl.BlockSpec((_BN, width), lambda i: (i, 0))


def _full_spec(rows, cols):
    return pl.BlockSpec((rows, cols), lambda i: (0, 0))


def _sc_segsum_body(src_hbm, dst_hbm, h_hbm, g_hbm, hsum_hbm, cred_hbm,
                    sb, db, asrc, adst, gbuf, acc, sem):
    cid = lax.axis_index("c")
    sid = lax.axis_index("s")
    izero16 = jnp.zeros((16,), jnp.int32)
    dummy16 = jnp.full((16,), _OWN, jnp.int32)
    fzero16 = jnp.zeros((16,), jnp.float32)

    def run(table, out):
        for r in range(2):
            lo = r * (16 * _OWN) + sid * _OWN

            def z_body(i, _):
                for j in range(16):
                    acc[pl.ds(i * 256 + j * 16, 16)] = fzero16
                return 0

            lax.fori_loop(0, _OWN + 8, z_body, 0)

            def blk(b, _):
                e0 = b * _BLK
                pltpu.sync_copy(src_hbm.at[pl.ds(e0, _BLK)], sb)
                pltpu.sync_copy(dst_hbm.at[pl.ds(e0, _BLK)], db)

                def pf(i, _):
                    row = i >> 3
                    colb = (i & 7) * 16
                    asrc[row, pl.ds(colb, 16)] = izero16
                    adst[row, pl.ds(colb, 16)] = dummy16
                    return 0

                lax.fori_loop(0, _ARB * 8, pf, 0)

                def cp(j, n):
                    o = j * 16
                    sv = sb[pl.ds(o, 16)]
                    dv = db[pl.ds(o, 16)]
                    dl = dv - lo
                    m = (dl >= 0) & (dl < _OWN)
                    cc = plsc.cumsum(m.astype(jnp.int32))
                    pos = (n - 1) + cc
                    plsc.store_scatter(asrc, [pos >> 7, pos & 127], sv,
                                       mask=m)
                    plsc.store_scatter(adst, [pos >> 7, pos & 127], dl,
                                       mask=m)
                    return n + jnp.max(cc)

                n = lax.fori_loop(0, _BLK // 16, cp, jnp.int32(0))

                def chunk(t, _):
                    pltpu.async_copy(table.at[asrc.at[t]], gbuf, sem).wait()

                    def grp(g, _):
                        didx = adst[t, pl.ds(g * 16, 16)]
                        for i in range(16):
                            dlw = didx[i] * 256
                            gr = g * 16 + i
                            for j in range(16):
                                plsc.addupdate(
                                    acc.at[pl.ds(dlw + j * 16, 16)],
                                    gbuf[gr, pl.ds(j * 16, 16)])
                        return 0

                    lax.fori_loop(0, 8, grp, 0)
                    return 0

                lax.fori_loop(0, (n + (_K - 1)) >> 7, chunk, 0)
                return 0

            lax.fori_loop(0, _NB, blk, 0)

            @pl.when(lo + _OWN <= _N)
            def _():
                pltpu.sync_copy(acc.at[pl.ds(0, _OWN * 256)],
                                out.at[pl.ds(lo * 256, _OWN * 256)])

            @pl.when(lo + _OWN > _N)
            def _():
                pltpu.sync_copy(acc.at[pl.ds(0, 80 * 256)],
                                out.at[pl.ds(lo * 256, 80 * 256)])

    @pl.when(cid == 0)
    def _():
        run(h_hbm, hsum_hbm)

    @pl.when(cid == 1)
    def _():
        run(g_hbm, cred_hbm)


def _sc_parent_body(src_hbm, dst_hbm, h_hbm, c_hbm, maxe_hbm, hp_hbm, cp_hbm,
                    src_v, dst_v, par_t, maxe_t, mslab, pslab, parm,
                    maxm, rowbuf, sp_par, sp_maxe, sem):
    cid = lax.axis_index("c")
    sid = lax.axis_index("s")

    @pl.when(cid == 0)
    def _():
        pltpu.sync_copy(src_hbm.at[pl.ds(sid * _EPT, _EPT)], src_v)
        pltpu.sync_copy(dst_hbm.at[pl.ds(sid * _EPT, _EPT)], dst_v)
        neg16 = jnp.full((16,), -1, jnp.int32)
        izero16 = jnp.zeros((16,), jnp.int32)

        def init_body(i, _):
            o = i * 16
            par_t[pl.ds(o, 16)] = izero16
            maxe_t[pl.ds(o, 16)] = neg16
            return 0

        lax.fori_loop(0, _N2 // 16, init_body, 0)

        iota = lax.iota(jnp.int32, 16)
        base0 = sid * _EPT

        def scan_body(j, _):
            o = j * 16
            sv = src_v[pl.ds(o, 16)]
            dv = dst_v[pl.ds(o, 16)]
            eid = (base0 + o) + iota
            _, keep = plsc.scan_count(sv)
            plsc.store_scatter(maxe_t, [sv], eid, mask=keep)
            plsc.store_scatter(par_t, [sv], dv, mask=keep)
            return 0

        lax.fori_loop(0, _EPT // 16, scan_body, 0)
        pltpu.sync_copy(par_t, sp_par.at[sid])
        pltpu.sync_copy(maxe_t, sp_maxe.at[sid])
        plsc.subcore_barrier()

        nb = sid * 640
        for t in range(16):
            pltpu.sync_copy(sp_maxe.at[t].at[pl.ds(nb, 640)], mslab.at[t])
            pltpu.sync_copy(sp_par.at[t].at[pl.ds(nb, 640)], pslab.at[t])

        def merge_body(v, _):
            o = v * 16
            bm = mslab[0, pl.ds(o, 16)]
            bp = pslab[0, pl.ds(o, 16)]
            for t in range(1, 16):
                m = mslab[t, pl.ds(o, 16)]
                p = pslab[t, pl.ds(o, 16)]
                upd = m > bm
                bm = jnp.where(upd, m, bm)
                bp = jnp.where(upd, p, bp)
            maxm[pl.ds(o, 16)] = bm
            parm[o >> 7, pl.ds(o & 127, 16)] = bp
            return 0

        lax.fori_loop(0, 640 // 16, merge_body, 0)

        pltpu.sync_copy(maxm, maxe_hbm.at[pl.ds(nb, 640)])

        for ch in range(5):
            gb = nb + ch * _K
            full = gb + _K <= _N
            part = (gb < _N) & (gb + _K > _N)
            pltpu.sync_copy(h_hbm.at[parm.at[ch]], rowbuf)

            @pl.when(full)
            def _():
                pltpu.sync_copy(rowbuf, hp_hbm.at[pl.ds(gb, _K)])

            @pl.when(part)
            def _():
                pltpu.sync_copy(rowbuf.at[pl.ds(0, 16)],
                                hp_hbm.at[pl.ds(gb, 16)])

            pltpu.sync_copy(c_hbm.at[parm.at[ch]], rowbuf)

            @pl.when(full)
            def _():
                pltpu.sync_copy(rowbuf, cp_hbm.at[pl.ds(gb, _K)])

            @pl.when(part)
            def _():
                pltpu.sync_copy(rowbuf.at[pl.ds(0, 16)],
                                cp_hbm.at[pl.ds(gb, 16)])


def _make_segsum(n, hs):
    f32, i32 = jnp.float32, jnp.int32
    mesh = plsc.VectorSubcoreMesh(core_axis_name="c", subcore_axis_name="s")
    return pl.kernel(
        _sc_segsum_body,
        out_type=[
            jax.ShapeDtypeStruct((n * hs,), f32),
            jax.ShapeDtypeStruct((n * hs,), f32),
        ],
        mesh=mesh,
        scratch_types=[
            pltpu.VMEM((_BLK,), i32),
            pltpu.VMEM((_BLK,), i32),
            pltpu.VMEM((_ARB, _K), i32),
            pltpu.VMEM((_ARB, _K), i32),
            pltpu.VMEM((_K, hs), f32),
            pltpu.VMEM(((_OWN + 8) * hs,), f32),
            pltpu.SemaphoreType.DMA,
        ],
        compiler_params=pltpu.CompilerParams(needs_layout_passes=False),
    )


def _make_parent(n, hs):
    f32, i32 = jnp.float32, jnp.int32
    mesh = plsc.VectorSubcoreMesh(core_axis_name="c", subcore_axis_name="s")
    return pl.kernel(
        _sc_parent_body,
        out_type=[
            jax.ShapeDtypeStruct((_N2,), i32),
            jax.ShapeDtypeStruct((n, hs), f32),
            jax.ShapeDtypeStruct((n, hs), f32),
        ],
        mesh=mesh,
        scratch_types=[
            pltpu.VMEM((_EPT,), i32),
            pltpu.VMEM((_EPT,), i32),
            pltpu.VMEM((_N2,), i32),
            pltpu.VMEM((_N2,), i32),
            pltpu.VMEM((16, 640), i32),
            pltpu.VMEM((16, 640), i32),
            pltpu.VMEM((5, _K), i32),
            pltpu.VMEM((640,), i32),
            pltpu.VMEM((_K, hs), f32),
            pltpu.VMEM_SHARED((16, _N2), i32),
            pltpu.VMEM_SHARED((16, _N2), i32),
            pltpu.SemaphoreType.DMA,
        ],
        compiler_params=pltpu.CompilerParams(needs_layout_passes=False),
    )


def kernel(x, h, c, edge_index, W_iou_bu, U_iou_bu, b_iou_bu, U_f_bu_W,
           U_f_bu_b, W_iou_td, U_iou_td, b_iou_td):
    n, xs = x.shape
    hs = h.shape[1]
    src = edge_index[0]
    dst = edge_index[1]
    grid = n // _BN
    f32 = jnp.float32

    g, pbu, ptd = pl.pallas_call(
        _tc_pre_body,
        grid=(grid,),
        in_specs=[
            _row_spec(hs), _row_spec(hs), _row_spec(xs),
            _full_spec(hs, hs), _full_spec(1, hs),
            _full_spec(xs, 3 * hs), _full_spec(1, 3 * hs),
            _full_spec(xs, 3 * hs), _full_spec(1, 3 * hs),
        ],
        out_specs=[_row_spec(hs), _row_spec(3 * hs), _row_spec(3 * hs)],
        out_shape=[
            jax.ShapeDtypeStruct((n, hs), f32),
            jax.ShapeDtypeStruct((n, 3 * hs), f32),
            jax.ShapeDtypeStruct((n, 3 * hs), f32),
        ],
    )(h, c, x, U_f_bu_W.T, U_f_bu_b.reshape(1, hs), W_iou_bu.T, b_iou_bu,
      W_iou_td.T, b_iou_td)

    hsum, cred = _make_segsum(n, hs)(src, dst, h, g)
    hsum = hsum.reshape(n, hs)
    cred = cred.reshape(n, hs)
    maxe, hp, cp = _make_parent(n, hs)(src, dst, h, c)

    out = pl.pallas_call(
        _tc_post_body,
        grid=(grid,),
        in_specs=[
            _row_spec(3 * hs), _row_spec(hs), _row_spec(hs), _row_spec(3 * hs),
            _row_spec(hs), _row_spec(hs),
            pl.BlockSpec((_BN, 1), lambda i: (i, 0)),
            _full_spec(hs, 3 * hs), _full_spec(hs, 3 * hs),
        ],
        out_specs=_row_spec(4 * hs),
        out_shape=jax.ShapeDtypeStruct((n, 4 * hs), f32),
    )(pbu, hsum, cred, ptd, hp, cp, maxe[:n].reshape(n, 1),
      U_iou_bu.T, U_iou_td.T)
    return out


# speedup vs baseline: 4.2719x; 4.2719x over previous
import functools

import jax
import jax.numpy as jnp
from jax import lax
from jax.experimental import pallas as pl
from jax.experimental.pallas import tpu as pltpu
from jax.experimental.pallas import tpu_sc as plsc

_N = 10000
_E = 160000
_HS = 256
_BN = 400
_EPT = _E // 16
_K = 128
_KS = 64
_BLK = 3200
_NB = _E // _BLK
_OWN = 320
_ARB = 52
_N2 = 10240


def _tc_pre_body(h_ref, c_ref, x_ref, uft_ref, bf_ref, wbut_ref, bbu_ref,
                 wtdt_ref, btd_ref, g_ref, pbu_ref, ptd_ref):
    f = jax.nn.sigmoid(
        jnp.dot(h_ref[...], uft_ref[...], preferred_element_type=jnp.float32)
        + bf_ref[...])
    g_ref[...] = f * c_ref[...]
    x = x_ref[...]
    pbu_ref[...] = jnp.dot(x, wbut_ref[...],
                           preferred_element_type=jnp.float32) + bbu_ref[...]
    ptd_ref[...] = jnp.dot(x, wtdt_ref[...],
                           preferred_element_type=jnp.float32) + btd_ref[...]


def _tc_post_body(pbu_ref, hsum_ref, cred_ref, ptd_ref, hp_ref, cp_ref,
                  maxe_ref, ubut_ref, utdt_ref, out_ref):
    iou_bu = pbu_ref[...] + jnp.dot(hsum_ref[...], ubut_ref[...],
                                    preferred_element_type=jnp.float32)
    i_bu = iou_bu[:, 0:_HS]
    o_bu = iou_bu[:, _HS:2 * _HS]
    u_bu = iou_bu[:, 2 * _HS:3 * _HS]
    c_bu = jax.nn.sigmoid(i_bu) * jnp.tanh(u_bu) + cred_ref[...]
    h_bu = jax.nn.sigmoid(o_bu) * jnp.tanh(c_bu)
    has = maxe_ref[...] >= 0
    hp = jnp.where(has, hp_ref[...], 0.0)
    cp = jnp.where(has, cp_ref[...], 0.0)
    iou_td = ptd_ref[...] + jnp.dot(hp, utdt_ref[...],
                                    preferred_element_type=jnp.float32)
    i_td = iou_td[:, 0:_HS]
    o_td = iou_td[:, _HS:2 * _HS]
    u_td = iou_td[:, 2 * _HS:3 * _HS]
    c_td = jax.nn.sigmoid(i_td) * jnp.tanh(u_td) + cp
    h_td = jax.nn.sigmoid(o_td) * jnp.tanh(c_td)
    out_ref[...] = jnp.concatenate([h_bu, c_bu, h_td, c_td], axis=1)


def _row_spec(width):
    return pl.BlockSpec((_BN, width), lambda i: (i, 0))


def _full_spec(rows, cols):
    return pl.BlockSpec((rows, cols), lambda i: (0, 0))


def _sc_segsum_body(src_hbm, dst_hbm, h_hbm, g_hbm, hsum_hbm, cred_hbm,
                    sba, dba, sbb, dbb, asrc, adst, gbuf, acc, sem, sga, sgb):
    cid = lax.axis_index("c")
    sid = lax.axis_index("s")
    izero16 = jnp.zeros((16,), jnp.int32)
    dummy16 = jnp.full((16,), _OWN, jnp.int32)
    fzero16 = jnp.zeros((16,), jnp.float32)
    iota = lax.iota(jnp.int32, 16)

    def start_stage(b, sb, db, sg):
        e0 = b * _BLK
        pltpu.async_copy(src_hbm.at[pl.ds(e0, _BLK)], sb, sg)
        pltpu.async_copy(dst_hbm.at[pl.ds(e0, _BLK)], db, sg)

    def wait_stage(sb, db, sg):
        pltpu.make_async_copy(src_hbm.at[pl.ds(0, _BLK)], sb, sg).wait()
        pltpu.make_async_copy(dst_hbm.at[pl.ds(0, _BLK)], db, sg).wait()

    def run(table, out):
        for r in range(2):
            lo = r * (16 * _OWN) + sid * _OWN

            def z_body(i, _):
                for j in range(16):
                    acc[pl.ds(i * 256 + j * 16, 16)] = fzero16
                return 0

            lax.fori_loop(0, _OWN + 8, z_body, 0)

            start_stage(0, sba, dba, sga)
            start_stage(1, sbb, dbb, sgb)

            def do_block(b, sb, db, sg):
                wait_stage(sb, db, sg)

                def cp(j, n):
                    o = j * 16
                    sv = sb[pl.ds(o, 16)]
                    dv = db[pl.ds(o, 16)]
                    dl = dv - lo
                    m = (dl >= 0) & (dl < _OWN)
                    cc = plsc.cumsum(m.astype(jnp.int32))
                    pos = (n - 1) + cc
                    plsc.store_scatter(asrc, [pos >> 6, pos & 63], sv,
                                       mask=m)
                    plsc.store_scatter(adst, [pos >> 6, pos & 63], dl,
                                       mask=m)
                    return n + jnp.max(cc)

                n = lax.fori_loop(0, _BLK // 16, cp, jnp.int32(0))

                @pl.when(b + 2 < _NB)
                def _():
                    start_stage(b + 2, sb, db, sg)

                rnd = ((n + 63) >> 6) << 6
                for k in range(4):
                    pos = n + k * 16 + iota
                    m = pos < rnd
                    plsc.store_scatter(asrc, [pos >> 6, pos & 63], izero16,
                                       mask=m)
                    plsc.store_scatter(adst, [pos >> 6, pos & 63], dummy16,
                                       mask=m)

                def chunk(t, _):
                    pltpu.async_copy(table.at[asrc.at[t]], gbuf, sem).wait()
                    gtrips = jnp.minimum(4, ((n - t * _KS) + 15) >> 4)

                    def grp(g, _):
                        didx = adst[t, pl.ds(g * 16, 16)]
                        for i in range(16):
                            dlw = didx[i] * 256
                            gr = g * 16 + i
                            for j in range(16):
                                plsc.addupdate(
                                    acc.at[pl.ds(dlw + j * 16, 16)],
                                    gbuf[gr, pl.ds(j * 16, 16)])
                        return 0

                    lax.fori_loop(0, gtrips, grp, 0)
                    return 0

                lax.fori_loop(0, (n + (_KS - 1)) >> 6, chunk, 0)

            def pair(p, _):
                do_block(2 * p, sba, dba, sga)
                do_block(2 * p + 1, sbb, dbb, sgb)
                return 0

            lax.fori_loop(0, _NB // 2, pair, 0)

            @pl.when(lo + _OWN <= _N)
            def _():
                pltpu.sync_copy(acc.at[pl.ds(0, _OWN * 256)],
                                out.at[pl.ds(lo * 256, _OWN * 256)])

            @pl.when(lo + _OWN > _N)
            def _():
                pltpu.sync_copy(acc.at[pl.ds(0, 80 * 256)],
                                out.at[pl.ds(lo * 256, 80 * 256)])

    @pl.when(cid == 0)
    def _():
        run(h_hbm, hsum_hbm)

    @pl.when(cid == 1)
    def _():
        run(g_hbm, cred_hbm)


def _sc_parent_body(src_hbm, dst_hbm, h_hbm, c_hbm, maxe_hbm, hp_hbm, cp_hbm,
                    src_v, dst_v, par_t, maxe_t, mslab, pslab, parm,
                    maxm, rowbuf, sp_par, sp_maxe, sem):
    cid = lax.axis_index("c")
    sid = lax.axis_index("s")

    @pl.when(cid == 0)
    def _():
        pltpu.sync_copy(src_hbm.at[pl.ds(sid * _EPT, _EPT)], src_v)
        pltpu.sync_copy(dst_hbm.at[pl.ds(sid * _EPT, _EPT)], dst_v)
        neg16 = jnp.full((16,), -1, jnp.int32)
        izero16 = jnp.zeros((16,), jnp.int32)

        def init_body(i, _):
            o = i * 16
            par_t[pl.ds(o, 16)] = izero16
            maxe_t[pl.ds(o, 16)] = neg16
            return 0

        lax.fori_loop(0, _N2 // 16, init_body, 0)

        iota = lax.iota(jnp.int32, 16)
        base0 = sid * _EPT

        def scan_body(j, _):
            o = j * 16
            sv = src_v[pl.ds(o, 16)]
            dv = dst_v[pl.ds(o, 16)]
            eid = (base0 + o) + iota
            _, keep = plsc.scan_count(sv)
            plsc.store_scatter(maxe_t, [sv], eid, mask=keep)
            plsc.store_scatter(par_t, [sv], dv, mask=keep)
            return 0

        lax.fori_loop(0, _EPT // 16, scan_body, 0)
        pltpu.sync_copy(par_t, sp_par.at[sid])
        pltpu.sync_copy(maxe_t, sp_maxe.at[sid])
        plsc.subcore_barrier()

        nb = sid * 640
        for t in range(16):
            pltpu.sync_copy(sp_maxe.at[t].at[pl.ds(nb, 640)], mslab.at[t])
            pltpu.sync_copy(sp_par.at[t].at[pl.ds(nb, 640)], pslab.at[t])

        def merge_body(v, _):
            o = v * 16
            bm = mslab[0, pl.ds(o, 16)]
            bp = pslab[0, pl.ds(o, 16)]
            for t in range(1, 16):
                m = mslab[t, pl.ds(o, 16)]
                p = pslab[t, pl.ds(o, 16)]
                upd = m > bm
                bm = jnp.where(upd, m, bm)
                bp = jnp.where(upd, p, bp)
            maxm[pl.ds(o, 16)] = bm
            parm[o >> 7, pl.ds(o & 127, 16)] = bp
            return 0

        lax.fori_loop(0, 640 // 16, merge_body, 0)

        pltpu.sync_copy(maxm, maxe_hbm.at[pl.ds(nb, 640)])

        for ch in range(5):
            gb = nb + ch * _K
            full = gb + _K <= _N
            part = (gb < _N) & (gb + _K > _N)
            pltpu.sync_copy(h_hbm.at[parm.at[ch]], rowbuf)

            @pl.when(full)
            def _():
                pltpu.sync_copy(rowbuf, hp_hbm.at[pl.ds(gb, _K)])

            @pl.when(part)
            def _():
                pltpu.sync_copy(rowbuf.at[pl.ds(0, 16)],
                                hp_hbm.at[pl.ds(gb, 16)])

            pltpu.sync_copy(c_hbm.at[parm.at[ch]], rowbuf)

            @pl.when(full)
            def _():
                pltpu.sync_copy(rowbuf, cp_hbm.at[pl.ds(gb, _K)])

            @pl.when(part)
            def _():
                pltpu.sync_copy(rowbuf.at[pl.ds(0, 16)],
                                cp_hbm.at[pl.ds(gb, 16)])


def _make_segsum(n, hs):
    f32, i32 = jnp.float32, jnp.int32
    mesh = plsc.VectorSubcoreMesh(core_axis_name="c", subcore_axis_name="s")
    return pl.kernel(
        _sc_segsum_body,
        out_type=[
            jax.ShapeDtypeStruct((n * hs,), f32),
            jax.ShapeDtypeStruct((n * hs,), f32),
        ],
        mesh=mesh,
        scratch_types=[
            pltpu.VMEM((_BLK,), i32),
            pltpu.VMEM((_BLK,), i32),
            pltpu.VMEM((_BLK,), i32),
            pltpu.VMEM((_BLK,), i32),
            pltpu.VMEM((_ARB, _KS), i32),
            pltpu.VMEM((_ARB, _KS), i32),
            pltpu.VMEM((_KS, hs), f32),
            pltpu.VMEM(((_OWN + 8) * hs,), f32),
            pltpu.SemaphoreType.DMA,
            pltpu.SemaphoreType.DMA,
            pltpu.SemaphoreType.DMA,
        ],
        compiler_params=pltpu.CompilerParams(needs_layout_passes=False),
    )


def _make_parent(n, hs):
    f32, i32 = jnp.float32, jnp.int32
    mesh = plsc.VectorSubcoreMesh(core_axis_name="c", subcore_axis_name="s")
    return pl.kernel(
        _sc_parent_body,
        out_type=[
            jax.ShapeDtypeStruct((_N2,), i32),
            jax.ShapeDtypeStruct((n, hs), f32),
            jax.ShapeDtypeStruct((n, hs), f32),
        ],
        mesh=mesh,
        scratch_types=[
            pltpu.VMEM((_EPT,), i32),
            pltpu.VMEM((_EPT,), i32),
            pltpu.VMEM((_N2,), i32),
            pltpu.VMEM((_N2,), i32),
            pltpu.VMEM((16, 640), i32),
            pltpu.VMEM((16, 640), i32),
            pltpu.VMEM((5, _K), i32),
            pltpu.VMEM((640,), i32),
            pltpu.VMEM((_K, hs), f32),
            pltpu.VMEM_SHARED((16, _N2), i32),
            pltpu.VMEM_SHARED((16, _N2), i32),
            pltpu.SemaphoreType.DMA,
        ],
        compiler_params=pltpu.CompilerParams(needs_layout_passes=False),
    )


def kernel(x, h, c, edge_index, W_iou_bu, U_iou_bu, b_iou_bu, U_f_bu_W,
           U_f_bu_b, W_iou_td, U_iou_td, b_iou_td):
    n, xs = x.shape
    hs = h.shape[1]
    src = edge_index[0]
    dst = edge_index[1]
    grid = n // _BN
    f32 = jnp.float32

    g, pbu, ptd = pl.pallas_call(
        _tc_pre_body,
        grid=(grid,),
        in_specs=[
            _row_spec(hs), _row_spec(hs), _row_spec(xs),
            _full_spec(hs, hs), _full_spec(1, hs),
            _full_spec(xs, 3 * hs), _full_spec(1, 3 * hs),
            _full_spec(xs, 3 * hs), _full_spec(1, 3 * hs),
        ],
        out_specs=[_row_spec(hs), _row_spec(3 * hs), _row_spec(3 * hs)],
        out_shape=[
            jax.ShapeDtypeStruct((n, hs), f32),
            jax.ShapeDtypeStruct((n, 3 * hs), f32),
            jax.ShapeDtypeStruct((n, 3 * hs), f32),
        ],
    )(h, c, x, U_f_bu_W.T, U_f_bu_b.reshape(1, hs), W_iou_bu.T, b_iou_bu,
      W_iou_td.T, b_iou_td)

    hsum, cred = _make_segsum(n, hs)(src, dst, h, g)
    hsum = hsum.reshape(n, hs)
    cred = cred.reshape(n, hs)
    maxe, hp, cp = _make_parent(n, hs)(src, dst, h, c)

    out = pl.pallas_call(
        _tc_post_body,
        grid=(grid,),
        in_specs=[
            _row_spec(3 * hs), _row_spec(hs), _row_spec(hs), _row_spec(3 * hs),
            _row_spec(hs), _row_spec(hs),
            pl.BlockSpec((_BN, 1), lambda i: (i, 0)),
            _full_spec(hs, 3 * hs), _full_spec(hs, 3 * hs),
        ],
        out_specs=_row_spec(4 * hs),
        out_shape=jax.ShapeDtypeStruct((n, 4 * hs), f32),
    )(pbu, hsum, cred, ptd, hp, cp, maxe[:n].reshape(n, 1),
      U_iou_bu.T, U_iou_td.T)
    return out
